# Initial kernel scaffold; baseline (speedup 1.0000x reference)
import jax
import jax.numpy as jnp
from jax.experimental import pallas as pl


def kernel(x, edge_index, W1, b1, W2, b2):
    raise NotImplementedError("write your pallas kernel here")


# speedup vs baseline: 4.7248x; 4.7248x over previous
import jax
import jax.numpy as jnp
from jax import lax
from jax.experimental import pallas as pl
from jax.experimental.pallas import tpu as pltpu
from jax.experimental.pallas import tpu_sc as plsc

N = 10000
NNZ = 320000
D = 128
NC = 2
NT = 16
L = 16

NPAD = 10112
R = 10240
RT = R // NT
CW = 64
CH = 160
SB = 16
NSLAB = CH // SB
PT = CH * CW
TOT = NC * NT * PT
KC = RT // 128
ZR = 16
DUMMY = N

_MESH = plsc.VectorSubcoreMesh(core_axis_name="c", subcore_axis_name="s")


def _inv(cnt):
    return jnp.where(cnt > 0.0, 1.0 / cnt, 0.0)


def _mm1_body(x_ref, w_ref, o_ref):
    o_ref[...] = jnp.dot(x_ref[...], w_ref[...],
                         preferred_element_type=jnp.float32)


def _mm1(xp, w):
    return pl.pallas_call(
        _mm1_body,
        out_shape=jax.ShapeDtypeStruct((NPAD, D), jnp.float32),
    )(xp, w)


def _combine_body(p_ref, cnt_ref, o_ref):
    scale = _inv(cnt_ref[0] + cnt_ref[1])
    o_ref[...] = scale * (p_ref[0] + p_ref[1])


def _combine(pacc, cnt2):
    return pl.pallas_call(
        _combine_body,
        out_shape=jax.ShapeDtypeStruct((NPAD, D), jnp.float32),
    )(pacc, cnt2)


def _mid_body(p_ref, cnt_ref, b_ref, w_ref, o_ref):
    scale = _inv(cnt_ref[0] + cnt_ref[1])
    h = scale * (p_ref[0] + p_ref[1]) + b_ref[...]
    h = jnp.maximum(h, 0.0)
    o_ref[...] = jnp.dot(h, w_ref[...], preferred_element_type=jnp.float32)


def _mid(pacc, cnt2, b, w):
    return pl.pallas_call(
        _mid_body,
        out_shape=jax.ShapeDtypeStruct((NPAD, D), jnp.float32),
    )(pacc, cnt2, b, w)


def _final_body(p_ref, cnt_ref, b_ref, o_ref):
    scale = _inv(cnt_ref[0] + cnt_ref[1])
    o_ref[...] = scale * (p_ref[0] + p_ref[1]) + b_ref[...]


def _final(pacc, cnt2, b):
    return pl.pallas_call(
        _final_body,
        out_shape=jax.ShapeDtypeStruct((NPAD, D), jnp.float32),
    )(pacc, cnt2, b)


@pl.kernel(
    out_type=(jax.ShapeDtypeStruct((NC, R), jnp.float32),
              jax.ShapeDtypeStruct((NC, R), jnp.float32)),
    mesh=_MESH,
    scratch_types=[
        pltpu.VMEM_SHARED((R,), jnp.float32),
        pltpu.VMEM_SHARED((R,), jnp.float32),
        pltpu.VMEM((1, SB, CW), jnp.int32),
        pltpu.VMEM((1, SB, CW), jnp.int32),
        pltpu.VMEM((CW,), jnp.float32),
        pltpu.VMEM((RT,), jnp.float32),
    ],
)
def _sc_counts(nidxh, eidxh, dcnt_h, bcnt_h,
               dcnt, bcnt, ign, ise, ones, zcnt):
    c = lax.axis_index("c")
    s = lax.axis_index("s")
    r0 = s * RT
    zf = jnp.zeros((L,), jnp.float32)
    one = jnp.ones((L,), jnp.float32)
    for q in range(CW // L):
        ones[pl.ds(q * L, L)] = one

    def zc(j, carry):
        zcnt[pl.ds(j * L, L)] = zf
        return carry
    lax.fori_loop(0, RT // L, zc, 0)
    pltpu.sync_copy(zcnt, dcnt.at[pl.ds(r0, RT)])
    pltpu.sync_copy(zcnt, bcnt.at[pl.ds(r0, RT)])
    plsc.subcore_barrier()

    def outer(q, carry):
        pltpu.sync_copy(nidxh.at[c, s, pl.ds(q * SB, SB)], ign.at[0])
        pltpu.sync_copy(eidxh.at[c, s, pl.ds(q * SB, SB)], ise.at[0])

        def cj(j, c2):
            pltpu.sync_copy(ones, dcnt.at[ign.at[0, j]], add=True)
            pltpu.sync_copy(ones, bcnt.at[ise.at[0, j]], add=True)
            return c2
        lax.fori_loop(0, SB, cj, 0)
        return carry
    lax.fori_loop(0, NSLAB, outer, 0)
    plsc.subcore_barrier()

    pltpu.sync_copy(dcnt.at[pl.ds(r0, RT)], dcnt_h.at[c, pl.ds(r0, RT)])
    pltpu.sync_copy(bcnt.at[pl.ds(r0, RT)], bcnt_h.at[c, pl.ds(r0, RT)])


def _make_sc_pass():
    scratch = [
        pltpu.VMEM_SHARED((R, D), jnp.float32),
        pltpu.VMEM((2, SB, CW), jnp.int32),
        pltpu.VMEM((2, SB, CW), jnp.int32),
        pltpu.VMEM((CW, D), jnp.float32),
        pltpu.VMEM((CW, D), jnp.float32),
        pltpu.VMEM((ZR, D), jnp.float32),
        pltpu.SemaphoreType.DMA,
        pltpu.SemaphoreType.DMA,
        pltpu.SemaphoreType.DMA,
        pltpu.SemaphoreType.DMA,
    ]

    def body(src_h, gidxh, sidxh, out_h,
             acc, ign, ise, g0, g1, zbuf, sem0, sem1, sem2, sem3):
        c = lax.axis_index("c")
        s = lax.axis_index("s")
        r0 = s * RT
        zf = jnp.zeros((L,), jnp.float32)

        def zb(j, carry):
            for q in range(D // L):
                zbuf[j, pl.ds(q * L, L)] = zf
            return carry
        lax.fori_loop(0, ZR, zb, 0)

        def zero_acc(j, carry):
            pltpu.sync_copy(zbuf, acc.at[pl.ds(r0 + j * ZR, ZR)])
            return carry
        lax.fori_loop(0, RT // ZR, zero_acc, 0)
        plsc.subcore_barrier()

        pltpu.sync_copy(gidxh.at[c, s, pl.ds(0, SB)], ign.at[0])
        pltpu.sync_copy(sidxh.at[c, s, pl.ds(0, SB)], ise.at[0])
        pltpu.async_copy(src_h.at[ign.at[0, 0]], g0, sem0)

        def outer(q, carry):
            pq = q % 2
            pn = (q + 1) % 2

            @pl.when(q + 1 < NSLAB)
            def _():
                pltpu.async_copy(gidxh.at[c, s, pl.ds((q + 1) * SB, SB)],
                                 ign.at[pn], sem2)
                pltpu.async_copy(sidxh.at[c, s, pl.ds((q + 1) * SB, SB)],
                                 ise.at[pn], sem3)

            def step(gb, sm, gbn, smn, j):
                pltpu.make_async_copy(src_h.at[ign.at[pq, j]], gb, sm).wait()

                @pl.when(j + 1 < SB)
                def _():
                    pltpu.async_copy(src_h.at[ign.at[pq, j + 1]], gbn, smn)

                @pl.when(jnp.logical_and(j + 1 == SB, q + 1 < NSLAB))
                def _():
                    pltpu.make_async_copy(
                        gidxh.at[c, s, pl.ds((q + 1) * SB, SB)],
                        ign.at[pn], sem2).wait()
                    pltpu.make_async_copy(
                        sidxh.at[c, s, pl.ds((q + 1) * SB, SB)],
                        ise.at[pn], sem3).wait()
                    pltpu.async_copy(src_h.at[ign.at[pn, 0]], gbn, smn)

                pltpu.sync_copy(gb, acc.at[ise.at[pq, j]], add=True)

            def inner(j, c2):
                @pl.when(j % 2 == 0)
                def _():
                    step(g0, sem0, g1, sem1, j)

                @pl.when(j % 2 == 1)
                def _():
                    step(g1, sem1, g0, sem0, j)
                return c2
            lax.fori_loop(0, SB, inner, 0)
            return carry
        lax.fori_loop(0, NSLAB, outer, 0)
        plsc.subcore_barrier()

        for k in range(KC):
            start = r0 + k * 128

            @pl.when(start < NPAD)
            def _():
                pltpu.sync_copy(acc.at[pl.ds(start, 128)],
                                out_h.at[c, pl.ds(start, 128)])

    return pl.kernel(
        body,
        out_type=jax.ShapeDtypeStruct((NC, NPAD, D), jnp.float32),
        mesh=_MESH,
        scratch_types=scratch,
    )


_sc_pass = _make_sc_pass()


@jax.jit
def kernel(x, edge_index, W1, b1, W2, b2):
    x = x.astype(jnp.float32)
    ei = edge_index.astype(jnp.int32)
    pad = jnp.full((TOT,), DUMMY, jnp.int32)
    nidx = pad.at[:NNZ].set(ei[0]).reshape(NC, NT, CH, CW)
    eidx = pad.at[:NNZ].set(ei[1]).reshape(NC, NT, CH, CW)
    xpad = jnp.zeros((NPAD, D), jnp.float32).at[:N].set(x)
    b1r = jnp.broadcast_to(b1.astype(jnp.float32), (1, D))
    b2r = jnp.broadcast_to(b2.astype(jnp.float32), (1, D))

    dcnt, bcnt = _sc_counts(nidx, eidx)
    dcnt2 = dcnt[:, :NPAD]
    bcnt2 = bcnt[:, :NPAD]
    dc = dcnt2.reshape(NC, NPAD, 1)
    bc = bcnt2.reshape(NC, NPAD, 1)

    xw1 = _mm1(xpad, W1.astype(jnp.float32))
    p1 = _sc_pass(xw1, nidx, eidx)
    ef1 = _combine(p1, bc)
    p2 = _sc_pass(ef1, eidx, nidx)
    xw2 = _mid(p2, dc, b1r, W2.astype(jnp.float32))
    p3 = _sc_pass(xw2, nidx, eidx)
    ef2 = _combine(p3, bc)
    p4 = _sc_pass(ef2, eidx, nidx)
    out = _final(p4, dc, b2r)
    return out[:N]


# speedup vs baseline: 6.3351x; 1.3408x over previous
import jax
import jax.numpy as jnp
from jax import lax
from jax.experimental import pallas as pl
from jax.experimental.pallas import tpu as pltpu
from jax.experimental.pallas import tpu_sc as plsc

N = 10000
NNZ = 320000
D = 128
NC = 2
NT = 16
L = 16

NPAD = 10112
R = 10240
RT = R // NT
CW = 128
CH = 80
SB = 8
NSLAB = CH // SB
PT = CH * CW
TOT = NC * NT * PT
KC = RT // 128
ZR = 16
DUMMY = N

_MESH = plsc.VectorSubcoreMesh(core_axis_name="c", subcore_axis_name="s")


def _inv(cnt):
    return jnp.where(cnt > 0.0, 1.0 / cnt, 0.0)


def _mm1_body(x_ref, w_ref, o_ref):
    o_ref[...] = jnp.dot(x_ref[...], w_ref[...],
                         preferred_element_type=jnp.float32)


def _mm1(xp, w):
    return pl.pallas_call(
        _mm1_body,
        out_shape=jax.ShapeDtypeStruct((NPAD, D), jnp.float32),
    )(xp, w)


def _combine_body(p_ref, cnt_ref, o_ref):
    scale = _inv(cnt_ref[0] + cnt_ref[1])
    o_ref[...] = scale * (p_ref[0] + p_ref[1])


def _combine(pacc, cnt2):
    return pl.pallas_call(
        _combine_body,
        out_shape=jax.ShapeDtypeStruct((NPAD, D), jnp.float32),
    )(pacc, cnt2)


def _mid_body(p_ref, cnt_ref, b_ref, w_ref, o_ref):
    scale = _inv(cnt_ref[0] + cnt_ref[1])
    h = scale * (p_ref[0] + p_ref[1]) + b_ref[...]
    h = jnp.maximum(h, 0.0)
    o_ref[...] = jnp.dot(h, w_ref[...], preferred_element_type=jnp.float32)


def _mid(pacc, cnt2, b, w):
    return pl.pallas_call(
        _mid_body,
        out_shape=jax.ShapeDtypeStruct((NPAD, D), jnp.float32),
    )(pacc, cnt2, b, w)


def _final_body(p_ref, cnt_ref, b_ref, o_ref):
    scale = _inv(cnt_ref[0] + cnt_ref[1])
    o_ref[...] = scale * (p_ref[0] + p_ref[1]) + b_ref[...]


def _final(pacc, cnt2, b):
    return pl.pallas_call(
        _final_body,
        out_shape=jax.ShapeDtypeStruct((NPAD, D), jnp.float32),
    )(pacc, cnt2, b)


@pl.kernel(
    out_type=(jax.ShapeDtypeStruct((NC, R), jnp.float32),
              jax.ShapeDtypeStruct((NC, R), jnp.float32)),
    mesh=_MESH,
    scratch_types=[
        pltpu.VMEM_SHARED((R,), jnp.float32),
        pltpu.VMEM_SHARED((R,), jnp.float32),
        pltpu.VMEM((1, SB, CW), jnp.int32),
        pltpu.VMEM((1, SB, CW), jnp.int32),
        pltpu.VMEM((CW,), jnp.float32),
        pltpu.VMEM((RT,), jnp.float32),
    ],
)
def _sc_counts(nidxh, eidxh, dcnt_h, bcnt_h,
               dcnt, bcnt, ign, ise, ones, zcnt):
    c = lax.axis_index("c")
    s = lax.axis_index("s")
    r0 = s * RT
    zf = jnp.zeros((L,), jnp.float32)
    one = jnp.ones((L,), jnp.float32)
    for q in range(CW // L):
        ones[pl.ds(q * L, L)] = one

    def zc(j, carry):
        zcnt[pl.ds(j * L, L)] = zf
        return carry
    lax.fori_loop(0, RT // L, zc, 0)
    pltpu.sync_copy(zcnt, dcnt.at[pl.ds(r0, RT)])
    pltpu.sync_copy(zcnt, bcnt.at[pl.ds(r0, RT)])
    plsc.subcore_barrier()

    def outer(q, carry):
        pltpu.sync_copy(nidxh.at[c, s, pl.ds(q * SB, SB)], ign.at[0])
        pltpu.sync_copy(eidxh.at[c, s, pl.ds(q * SB, SB)], ise.at[0])

        def cj(j, c2):
            pltpu.sync_copy(ones, dcnt.at[ign.at[0, j]], add=True)
            pltpu.sync_copy(ones, bcnt.at[ise.at[0, j]], add=True)
            return c2
        lax.fori_loop(0, SB, cj, 0)
        return carry
    lax.fori_loop(0, NSLAB, outer, 0)
    plsc.subcore_barrier()

    pltpu.sync_copy(dcnt.at[pl.ds(r0, RT)], dcnt_h.at[c, pl.ds(r0, RT)])
    pltpu.sync_copy(bcnt.at[pl.ds(r0, RT)], bcnt_h.at[c, pl.ds(r0, RT)])


def _make_sc_pass():
    scratch = [
        pltpu.VMEM_SHARED((R, D), jnp.float32),
        pltpu.VMEM((2, SB, CW), jnp.int32),
        pltpu.VMEM((2, SB, CW), jnp.int32),
        pltpu.VMEM((CW, D), jnp.float32),
        pltpu.VMEM((CW, D), jnp.float32),
        pltpu.VMEM((ZR, D), jnp.float32),
        pltpu.SemaphoreType.DMA,
        pltpu.SemaphoreType.DMA,
        pltpu.SemaphoreType.DMA,
        pltpu.SemaphoreType.DMA,
        pltpu.SemaphoreType.DMA,
        pltpu.SemaphoreType.DMA,
    ]

    def body(src_h, gidxh, sidxh, out_h,
             acc, ign, ise, g0, g1, zbuf,
             sem0, sem1, sem2, sem3, ssem0, ssem1):
        c = lax.axis_index("c")
        s = lax.axis_index("s")
        r0 = s * RT
        zf = jnp.zeros((L,), jnp.float32)

        def zb(j, carry):
            for q in range(D // L):
                zbuf[j, pl.ds(q * L, L)] = zf
            return carry
        lax.fori_loop(0, ZR, zb, 0)

        def zero_acc(j, carry):
            pltpu.sync_copy(zbuf, acc.at[pl.ds(r0 + j * ZR, ZR)])
            return carry
        lax.fori_loop(0, RT // ZR, zero_acc, 0)
        plsc.subcore_barrier()

        pltpu.sync_copy(gidxh.at[c, s, pl.ds(0, SB)], ign.at[0])
        pltpu.sync_copy(sidxh.at[c, s, pl.ds(0, SB)], ise.at[0])
        pltpu.async_copy(src_h.at[ign.at[0, 0]], g0, sem0)

        def outer(q, carry):
            pq = q % 2
            pn = (q + 1) % 2

            def wait_scatter(buf, ssm, j):
                pltpu.make_async_copy(buf, acc.at[ise.at[pq, j]], ssm).wait()

            def step(gb, gsm, ssm, gbn, gsmn, ssmn, j):
                pltpu.make_async_copy(src_h.at[ign.at[pq, j]], gb, gsm).wait()

                @pl.when(jnp.logical_and(j == 1, q + 1 < NSLAB))
                def _():
                    pltpu.async_copy(gidxh.at[c, s, pl.ds((q + 1) * SB, SB)],
                                     ign.at[pn], sem2)
                    pltpu.async_copy(sidxh.at[c, s, pl.ds((q + 1) * SB, SB)],
                                     ise.at[pn], sem3)

                @pl.when(j + 1 < SB)
                def _():
                    @pl.when(jnp.logical_or(q > 0, j > 0))
                    def _():
                        wait_scatter(gbn, ssmn, j)
                    pltpu.async_copy(src_h.at[ign.at[pq, j + 1]], gbn, gsmn)

                @pl.when(jnp.logical_and(j + 1 == SB, q + 1 < NSLAB))
                def _():
                    wait_scatter(gbn, ssmn, j)
                    pltpu.make_async_copy(
                        gidxh.at[c, s, pl.ds((q + 1) * SB, SB)],
                        ign.at[pn], sem2).wait()
                    pltpu.make_async_copy(
                        sidxh.at[c, s, pl.ds((q + 1) * SB, SB)],
                        ise.at[pn], sem3).wait()
                    pltpu.async_copy(src_h.at[ign.at[pn, 0]], gbn, gsmn)

                pltpu.async_copy(gb, acc.at[ise.at[pq, j]], ssm, add=True)

            def inner(j, c2):
                @pl.when(j % 2 == 0)
                def _():
                    step(g0, sem0, ssem0, g1, sem1, ssem1, j)

                @pl.when(j % 2 == 1)
                def _():
                    step(g1, sem1, ssem1, g0, sem0, ssem0, j)
                return c2
            lax.fori_loop(0, SB, inner, 0)
            return carry
        lax.fori_loop(0, NSLAB, outer, 0)

        pltpu.make_async_copy(g0, acc.at[ise.at[(NSLAB - 1) % 2, SB - 2]],
                              ssem0).wait()
        pltpu.make_async_copy(g1, acc.at[ise.at[(NSLAB - 1) % 2, SB - 1]],
                              ssem1).wait()
        plsc.subcore_barrier()

        for k in range(KC):
            start = r0 + k * 128

            @pl.when(start < NPAD)
            def _():
                pltpu.sync_copy(acc.at[pl.ds(start, 128)],
                                out_h.at[c, pl.ds(start, 128)])

    return pl.kernel(
        body,
        out_type=jax.ShapeDtypeStruct((NC, NPAD, D), jnp.float32),
        mesh=_MESH,
        scratch_types=scratch,
    )


_sc_pass = _make_sc_pass()


@jax.jit
def kernel(x, edge_index, W1, b1, W2, b2):
    x = x.astype(jnp.float32)
    ei = edge_index.astype(jnp.int32)
    pad = jnp.full((TOT,), DUMMY, jnp.int32)
    nidx = pad.at[:NNZ].set(ei[0]).reshape(NC, NT, CH, CW)
    eidx = pad.at[:NNZ].set(ei[1]).reshape(NC, NT, CH, CW)
    xpad = jnp.zeros((NPAD, D), jnp.float32).at[:N].set(x)
    b1r = jnp.broadcast_to(b1.astype(jnp.float32), (1, D))
    b2r = jnp.broadcast_to(b2.astype(jnp.float32), (1, D))

    dcnt, bcnt = _sc_counts(nidx, eidx)
    dcnt2 = dcnt[:, :NPAD]
    bcnt2 = bcnt[:, :NPAD]
    dc = dcnt2.reshape(NC, NPAD, 1)
    bc = bcnt2.reshape(NC, NPAD, 1)

    xw1 = _mm1(xpad, W1.astype(jnp.float32))
    p1 = _sc_pass(xw1, nidx, eidx)
    ef1 = _combine(p1, bc)
    p2 = _sc_pass(ef1, eidx, nidx)
    xw2 = _mid(p2, dc, b1r, W2.astype(jnp.float32))
    p3 = _sc_pass(xw2, nidx, eidx)
    ef2 = _combine(p3, bc)
    p4 = _sc_pass(ef2, eidx, nidx)
    out = _final(p4, dc, b2r)
    return out[:N]
